# baseline (speedup 1.0000x reference)
import functools

import jax
import jax.numpy as jnp
from jax import lax
from jax.experimental import pallas as pl
from jax.experimental.pallas import tpu as pltpu
from jax.experimental.pallas import tpu_sc as plsc

L = 16
E = 16
NS = 16


def _sc_forward():
    B = 16384
    R = B // NS

    mesh = plsc.VectorSubcoreMesh(core_axis_name="c", subcore_axis_name="s",
                                  num_cores=1)

    @functools.partial(
        pl.kernel,
        out_type=jax.ShapeDtypeStruct((B,), jnp.float32),
        mesh=mesh,
        compiler_params=pltpu.CompilerParams(use_tc_tiling_on_sc=False),
        scratch_types=[
            pltpu.VMEM((R,), jnp.int32),
            pltpu.VMEM((R,), jnp.int32),
            pltpu.VMEM((E, R), jnp.float32),
            pltpu.VMEM((E, R), jnp.float32),
            pltpu.VMEM((R,), jnp.float32),
            pltpu.VMEM((R,), jnp.float32),
            pltpu.VMEM((R,), jnp.float32),
            pltpu.VMEM((L,), jnp.float32),
            pltpu.VMEM((NS, L), jnp.float32),
            pltpu.VMEM_SHARED((NS, L), jnp.float32),
            pltpu.SemaphoreType.DMA,
        ],
    )
    def body(uid_h, iid_h, uet_h, ub_h, iet_h, ib_h, out_h,
             uid_v, iid_v, du, di, ubv, ibv, outv, accv, allp, shr, sem):
        sid = lax.axis_index("s")
        base = sid * R

        pltpu.sync_copy(uid_h.at[pl.ds(base, R)], uid_v)
        pltpu.sync_copy(iid_h.at[pl.ds(base, R)], iid_v)

        copies = []
        for e in range(E):
            copies.append(pltpu.async_copy(
                uet_h.at[e].at[uid_v], du.at[e], sem))
            copies.append(pltpu.async_copy(
                iet_h.at[e].at[iid_v], di.at[e], sem))
        copies.append(pltpu.async_copy(ub_h.at[uid_v], ubv, sem))
        copies.append(pltpu.async_copy(ib_h.at[iid_v], ibv, sem))
        for cp in copies:
            cp.wait()

        def dot_e(e):
            def dot_g(g, acc):
                return acc + (du[e, pl.ds(g * L, L)]
                              * di[e, pl.ds(g * L, L)])
            return lax.fori_loop(0, R // L, dot_g,
                                 jnp.zeros((L,), jnp.float32))

        acc = dot_e(0)
        for e in range(1, E):
            acc = acc + dot_e(e)
        accv[...] = acc

        pltpu.sync_copy(accv, shr.at[sid])
        plsc.subcore_barrier()
        pltpu.sync_copy(shr, allp)
        tot = allp[0]
        for j in range(1, NS):
            tot = tot + allp[j]
        lanes = lax.iota(jnp.int32, L)
        for shift in (1, 2, 4, 8):
            tot = tot + tot.at[(lanes + shift) % L].get(
                mode="promise_in_bounds")
        s = tot

        def out_g(k, _):
            x = s + ubv[pl.ds(k * L, L)] + ibv[pl.ds(k * L, L)]
            outv[pl.ds(k * L, L)] = 1.0 / (1.0 + jnp.exp(-x))
            return 0

        lax.fori_loop(0, R // L, out_g, 0)
        pltpu.sync_copy(outv, out_h.at[pl.ds(base, R)])

    return body


def kernel(inputs, user_embedding, user_bias, item_embedding, item_bias):
    B = inputs.shape[0]
    ii = inputs.astype(jnp.int32)
    uid = ii[:, 0]
    iid = ii[:, 1]
    uet = user_embedding.T
    iet = item_embedding.T
    ub = user_bias.reshape(-1)
    ib = item_bias.reshape(-1)
    fwd = _sc_forward()
    out = fwd(uid, iid, uet, ub, iet, ib)
    return out.reshape(B, 1)
